# baseline (speedup 1.0000x reference)
import jax
import jax.numpy as jnp
from jax.experimental import pallas as pl
from jax.experimental.pallas import tpu as pltpu

_SCALE = 1.0 / 32
_P = 7


def _roi_pool_kernel(bidx_ref, valid_ref, x0_ref, x1_ref,
                     feat_ref, cmask_ref, rmask_ref, out_ref, s1_ref):
    r = pl.program_id(0)
    b = bidx_ref[r]
    gref = feat_ref.at[b]
    cm = cmask_ref[0]
    rm = rmask_ref[0]
    v = valid_ref[r]
    h0 = x0_ref[r]
    h1 = x1_ref[r]

    @pl.when(r == 0)
    def _init():
        s1_ref[...] = jnp.full_like(s1_ref, -jnp.inf)

    for h in range(64):
        @pl.when((h >= h0) & (h < h1))
        def _row():
            row = gref[h]
            for j in range(_P):
                s1_ref[j, h] = jnp.max(row + cm[j][:, None], axis=0)

    for i in range(_P):
        rmi = rm[i][:, None]
        for j in range(_P):
            val = jnp.max(s1_ref[j] + rmi, axis=0)
            out_ref[0, i, j] = jnp.where(v > 0, val, 0.0)


def kernel(features, rois):
    N, C, H, W = features.shape
    R = rois.shape[0]
    P = _P

    bidx = rois[:, 0].astype(jnp.int32)
    bbox = jnp.round(rois[:, 1:] * _SCALE).astype(jnp.int32)
    x0 = jnp.clip(bbox[:, 0], 0, W - 1)
    y0 = jnp.clip(bbox[:, 1], 0, H - 1)
    x1 = jnp.clip(bbox[:, 2], 0, W - 1)
    y1 = jnp.clip(bbox[:, 3], 0, H - 1)
    valid = ((x0 < x1) & (y0 < y1)).astype(jnp.int32)
    Lh = x1 - x0
    Lw = y1 - y0

    j = jnp.arange(P)
    neg = jnp.float32(-jnp.inf)
    ws = y0[:, None] + (j[None, :] * Lw[:, None]) // P
    we = y0[:, None] + ((j[None, :] + 1) * Lw[:, None] + P - 1) // P
    idx_w = jnp.arange(W)
    cmask = jnp.where(
        (idx_w[None, None, :] >= ws[:, :, None]) & (idx_w[None, None, :] < we[:, :, None]),
        jnp.float32(0), neg,
    )
    hs = x0[:, None] + (j[None, :] * Lh[:, None]) // P
    he = x0[:, None] + ((j[None, :] + 1) * Lh[:, None] + P - 1) // P
    idx_h = jnp.arange(H)
    rmask = jnp.where(
        (idx_h[None, None, :] >= hs[:, :, None]) & (idx_h[None, None, :] < he[:, :, None]),
        jnp.float32(0), neg,
    )

    fhwc = jnp.transpose(features, (0, 2, 3, 1))

    out = pl.pallas_call(
        _roi_pool_kernel,
        grid_spec=pltpu.PrefetchScalarGridSpec(
            num_scalar_prefetch=4,
            grid=(R,),
            in_specs=[
                pl.BlockSpec((N, H, W, C), lambda r, *_: (0, 0, 0, 0)),
                pl.BlockSpec((1, P, W), lambda r, *_: (r, 0, 0)),
                pl.BlockSpec((1, P, H), lambda r, *_: (r, 0, 0)),
            ],
            out_specs=pl.BlockSpec((1, P, P, C), lambda r, *_: (r, 0, 0, 0)),
            scratch_shapes=[pltpu.VMEM((P, H, C), jnp.float32)],
        ),
        out_shape=jax.ShapeDtypeStruct((R, P, P, C), jnp.float32),
        compiler_params=pltpu.CompilerParams(
            dimension_semantics=("arbitrary",),
            vmem_limit_bytes=60 * 1024 * 1024,
        ),
        name="roi_pool",
    )(bidx, valid, x0, x1, fhwc, cmask, rmask)

    return jnp.transpose(out, (0, 3, 1, 2))


# speedup vs baseline: 1.6390x; 1.6390x over previous
import jax
import jax.numpy as jnp
from jax.experimental import pallas as pl
from jax.experimental.pallas import tpu as pltpu

_SCALE = 1.0 / 32
_P = 7


def _roi_pool_kernel(bidx_ref, valid_ref, x0_ref, x1_ref,
                     feat_ref, cmask_ref, rmask_ref, out_ref, s1_ref):
    r = pl.program_id(0)
    b = bidx_ref[r]
    gref = feat_ref.at[b]
    cm = cmask_ref[0]
    rm = rmask_ref[0]
    v = valid_ref[r]
    h0 = x0_ref[r]
    h1 = x1_ref[r]

    @pl.when(r == 0)
    def _init():
        s1_ref[...] = jnp.full_like(s1_ref, -jnp.inf)

    for hb in range(8):
        @pl.when((hb * 8 + 8 > h0) & (hb * 8 < h1))
        def _chunk():
            for h in range(hb * 8, hb * 8 + 8):
                row = gref[h]
                for j in range(_P):
                    s1_ref[j, h] = jnp.max(row + cm[j][:, None], axis=0)

    for i in range(_P):
        rmi = rm[i][:, None]
        for j in range(_P):
            val = jnp.max(s1_ref[j] + rmi, axis=0)
            out_ref[0, i, j] = jnp.where(v > 0, val, 0.0)


def kernel(features, rois):
    N, C, H, W = features.shape
    R = rois.shape[0]
    P = _P

    bidx = rois[:, 0].astype(jnp.int32)
    bbox = jnp.round(rois[:, 1:] * _SCALE).astype(jnp.int32)
    x0 = jnp.clip(bbox[:, 0], 0, W - 1)
    y0 = jnp.clip(bbox[:, 1], 0, H - 1)
    x1 = jnp.clip(bbox[:, 2], 0, W - 1)
    y1 = jnp.clip(bbox[:, 3], 0, H - 1)
    valid = ((x0 < x1) & (y0 < y1)).astype(jnp.int32)
    Lh = x1 - x0
    Lw = y1 - y0

    j = jnp.arange(P)
    neg = jnp.float32(-jnp.inf)
    ws = y0[:, None] + (j[None, :] * Lw[:, None]) // P
    we = y0[:, None] + ((j[None, :] + 1) * Lw[:, None] + P - 1) // P
    idx_w = jnp.arange(W)
    cmask = jnp.where(
        (idx_w[None, None, :] >= ws[:, :, None]) & (idx_w[None, None, :] < we[:, :, None]),
        jnp.float32(0), neg,
    )
    hs = x0[:, None] + (j[None, :] * Lh[:, None]) // P
    he = x0[:, None] + ((j[None, :] + 1) * Lh[:, None] + P - 1) // P
    idx_h = jnp.arange(H)
    rmask = jnp.where(
        (idx_h[None, None, :] >= hs[:, :, None]) & (idx_h[None, None, :] < he[:, :, None]),
        jnp.float32(0), neg,
    )

    fhwc = jnp.transpose(features, (0, 2, 3, 1))

    out = pl.pallas_call(
        _roi_pool_kernel,
        grid_spec=pltpu.PrefetchScalarGridSpec(
            num_scalar_prefetch=4,
            grid=(R,),
            in_specs=[
                pl.BlockSpec((N, H, W, C), lambda r, *_: (0, 0, 0, 0)),
                pl.BlockSpec((1, P, W), lambda r, *_: (r, 0, 0)),
                pl.BlockSpec((1, P, H), lambda r, *_: (r, 0, 0)),
            ],
            out_specs=pl.BlockSpec((1, P, P, C), lambda r, *_: (r, 0, 0, 0)),
            scratch_shapes=[pltpu.VMEM((P, H, C), jnp.float32)],
        ),
        out_shape=jax.ShapeDtypeStruct((R, P, P, C), jnp.float32),
        compiler_params=pltpu.CompilerParams(
            dimension_semantics=("arbitrary",),
            vmem_limit_bytes=60 * 1024 * 1024,
        ),
        name="roi_pool",
    )(bidx, valid, x0, x1, fhwc, cmask, rmask)

    return jnp.transpose(out, (0, 3, 1, 2))
